# Initial kernel scaffold; baseline (speedup 1.0000x reference)
import jax
import jax.numpy as jnp
from jax.experimental import pallas as pl


def kernel(x, dw_w, dw_b, ln_g, ln_b, w1, b1, w2, b2, gamma):
    raise NotImplementedError("write your pallas kernel here")


# speedup vs baseline: 1.9376x; 1.9376x over previous
import functools

import jax
import jax.numpy as jnp
from jax.experimental import pallas as pl
from jax.experimental.pallas import tpu as pltpu

_H = 56
_W = 56
_C = 128
_F = 512
_EPS = 1e-6
_STRIP = 8
_NSTRIP = _H // _STRIP


def _body(x_ref, taps_ref, dwb_ref, lng_ref, lnb_ref, w1_ref, b1_ref,
          w2_ref, b2_ref, gamma_ref, o_ref, pad_ref):
    pad_ref[...] = jnp.zeros((_H + 6, _W + 6, _C), jnp.float32)
    pad_ref[3:3 + _H, 3:3 + _W, :] = x_ref[0]

    def strip(s, _):
        h0 = s * _STRIP
        acc = jnp.zeros((_STRIP, _W, _C), jnp.float32) + dwb_ref[...]
        for i in range(7):
            rows = pad_ref[pl.ds(h0 + i, _STRIP)]
            for j in range(7):
                acc = acc + rows[:, j:j + _W, :] * taps_ref[7 * i + j]
        mu = jnp.mean(acc, axis=-1, keepdims=True)
        d = acc - mu
        var = jnp.mean(d * d, axis=-1, keepdims=True)
        yn = d * jax.lax.rsqrt(var + _EPS) * lng_ref[...] + lnb_ref[...]
        yn2 = yn.reshape(_STRIP * _W, _C)
        h1 = jnp.dot(yn2, w1_ref[...], preferred_element_type=jnp.float32)
        h1 = h1 + b1_ref[...]
        h1 = 0.5 * h1 * (1.0 + jax.lax.erf(h1 * 0.7071067811865476))
        y2 = jnp.dot(h1, w2_ref[...], preferred_element_type=jnp.float32)
        y2 = (y2 + b2_ref[...]) * gamma_ref[...]
        o_ref[0, pl.ds(h0, _STRIP)] = (
            x_ref[0, pl.ds(h0, _STRIP)] + y2.reshape(_STRIP, _W, _C))
        return ()

    jax.lax.fori_loop(0, _NSTRIP, strip, (), unroll=False)


@jax.jit
def kernel(x, dw_w, dw_b, ln_g, ln_b, w1, b1, w2, b2, gamma):
    n = x.shape[0]
    xt = jnp.transpose(x, (0, 2, 3, 1))
    taps = jnp.transpose(dw_w[:, 0, :, :], (1, 2, 0)).reshape(49, _C)
    row = lambda v: v.reshape(1, -1)
    out_nhwc = pl.pallas_call(
        _body,
        grid=(n,),
        in_specs=[
            pl.BlockSpec((1, _H, _W, _C), lambda b: (b, 0, 0, 0)),
            pl.BlockSpec((49, _C), lambda b: (0, 0)),
            pl.BlockSpec((1, _C), lambda b: (0, 0)),
            pl.BlockSpec((1, _C), lambda b: (0, 0)),
            pl.BlockSpec((1, _C), lambda b: (0, 0)),
            pl.BlockSpec((_C, _F), lambda b: (0, 0)),
            pl.BlockSpec((1, _F), lambda b: (0, 0)),
            pl.BlockSpec((_F, _C), lambda b: (0, 0)),
            pl.BlockSpec((1, _C), lambda b: (0, 0)),
            pl.BlockSpec((1, _C), lambda b: (0, 0)),
        ],
        out_specs=pl.BlockSpec((1, _H, _W, _C), lambda b: (b, 0, 0, 0)),
        out_shape=jax.ShapeDtypeStruct((n, _H, _W, _C), jnp.float32),
        scratch_shapes=[pltpu.VMEM((_H + 6, _W + 6, _C), jnp.float32)],
        compiler_params=pltpu.CompilerParams(
            dimension_semantics=("parallel",),
            vmem_limit_bytes=48 * 1024 * 1024,
        ),
        name="convnext_block",
    )(xt, taps, row(dw_b), row(ln_g), row(ln_b), w1, row(b1), w2, row(b2),
      row(gamma))
    return jnp.transpose(out_nhwc, (0, 3, 1, 2))


# speedup vs baseline: 2.2872x; 1.1804x over previous
import jax
import jax.numpy as jnp
from jax.experimental import pallas as pl
from jax.experimental.pallas import tpu as pltpu

_H = 56
_W = 56
_WP = 64
_C = 128
_F = 512
_EPS = 1e-6
_STRIP = 8
_NSTRIP = _H // _STRIP


def _body(x_ref, taps_ref, dwb_ref, lng_ref, lnb_ref, w1_ref, b1_ref,
          w2_ref, b2_ref, gamma_ref, o_ref, pad_ref, shb_ref):
    pad_ref[...] = jnp.zeros((_H + 6, _WP + 8, _C), jnp.float32)
    pad_ref[3:3 + _H, 3:3 + _W, :] = x_ref[0]
    for j in range(7):
        shb_ref[j] = pad_ref[:, j:j + _WP, :].astype(jnp.bfloat16)

    def strip(s, _):
        h0 = s * _STRIP
        acc = jnp.zeros((_STRIP, _WP, _C), jnp.bfloat16)
        for j in range(7):
            for i in range(7):
                acc = acc + shb_ref[j, pl.ds(h0 + i, _STRIP)] * taps_ref[7 * i + j]
        y = acc[:, :_W, :].astype(jnp.float32) + dwb_ref[...]
        mu = jnp.mean(y, axis=-1, keepdims=True)
        d = y - mu
        var = jnp.mean(d * d, axis=-1, keepdims=True)
        yn = d * jax.lax.rsqrt(var + _EPS) * lng_ref[...] + lnb_ref[...]
        yn2 = yn.reshape(_STRIP * _W, _C).astype(jnp.bfloat16)
        h1 = jnp.dot(yn2, w1_ref[...], preferred_element_type=jnp.float32)
        h1 = h1 + b1_ref[...]
        h1 = 0.5 * h1 * (1.0 + jax.lax.erf(h1 * 0.7071067811865476))
        y2 = jnp.dot(h1.astype(jnp.bfloat16), w2_ref[...],
                     preferred_element_type=jnp.float32)
        y2 = (y2 + b2_ref[...]) * gamma_ref[...]
        o_ref[0, pl.ds(h0, _STRIP)] = (
            x_ref[0, pl.ds(h0, _STRIP)] + y2.reshape(_STRIP, _W, _C))
        return ()

    jax.lax.fori_loop(0, _NSTRIP, strip, (), unroll=False)


@jax.jit
def kernel(x, dw_w, dw_b, ln_g, ln_b, w1, b1, w2, b2, gamma):
    n = x.shape[0]
    xt = jnp.transpose(x, (0, 2, 3, 1))
    taps = jnp.transpose(dw_w[:, 0, :, :], (1, 2, 0)).reshape(49, _C)
    taps = taps.astype(jnp.bfloat16)
    row = lambda v: v.reshape(1, -1)
    out_nhwc = pl.pallas_call(
        _body,
        grid=(n,),
        in_specs=[
            pl.BlockSpec((1, _H, _W, _C), lambda b: (b, 0, 0, 0)),
            pl.BlockSpec((49, _C), lambda b: (0, 0)),
            pl.BlockSpec((1, _C), lambda b: (0, 0)),
            pl.BlockSpec((1, _C), lambda b: (0, 0)),
            pl.BlockSpec((1, _C), lambda b: (0, 0)),
            pl.BlockSpec((_C, _F), lambda b: (0, 0)),
            pl.BlockSpec((1, _F), lambda b: (0, 0)),
            pl.BlockSpec((_F, _C), lambda b: (0, 0)),
            pl.BlockSpec((1, _C), lambda b: (0, 0)),
            pl.BlockSpec((1, _C), lambda b: (0, 0)),
        ],
        out_specs=pl.BlockSpec((1, _H, _W, _C), lambda b: (b, 0, 0, 0)),
        out_shape=jax.ShapeDtypeStruct((n, _H, _W, _C), jnp.float32),
        scratch_shapes=[
            pltpu.VMEM((_H + 6, _WP + 8, _C), jnp.float32),
            pltpu.VMEM((7, _H + 6, _WP, _C), jnp.bfloat16),
        ],
        compiler_params=pltpu.CompilerParams(
            dimension_semantics=("parallel",),
            vmem_limit_bytes=48 * 1024 * 1024,
        ),
        name="convnext_block",
    )(xt, taps, row(dw_b), row(ln_g), row(ln_b), w1.astype(jnp.bfloat16),
      row(b1), w2.astype(jnp.bfloat16), row(b2), row(gamma))
    return jnp.transpose(out_nhwc, (0, 3, 1, 2))
